# baseline (speedup 1.0000x reference)
import jax
import jax.numpy as jnp
import numpy as np
from jax import lax
from jax.experimental import pallas as pl
from jax.experimental.pallas import tpu as pltpu
from jax.experimental.pallas import tpu_sc as plsc

N = 10000
E = 320000
D = 128
MASK_RATIO = 0.5

NC = 2
NS = 16
NW = NC * NS
EPW = E // NW
G = 80
CHUNKS = EPW // G
NPAD = 10240
RPT = NPAD // NS
NBUF = 3
ZROWS = 128

with jax.default_device(jax.devices("cpu")[0]):
    _MASK_F32 = np.asarray(
        jax.random.uniform(jax.random.key(42), (N, D)) < MASK_RATIO,
        dtype=np.float32,
    )


def _mesh():
    return plsc.VectorSubcoreMesh(
        core_axis_name="c", subcore_axis_name="s", num_cores=NC, num_subcores=NS
    )


def _fill_rows(ref, nrows, ncols, value):
    vec = jnp.full((16,), value, jnp.float32)

    @pl.loop(0, nrows)
    def _(r):
        @pl.loop(0, ncols // 16)
        def _(q):
            ref[r, pl.ds(q * 16, 16)] = vec


def _agg_body(x_hbm, src_hbm, dst_hbm, out_hbm, srcv, dstv, rowsv, zbuf,
              acc_sh, sems):
    cid = lax.axis_index("c")
    sid = lax.axis_index("s")
    w = cid * NS + sid

    _fill_rows(zbuf, ZROWS, D, 0.0)

    @pl.loop(0, RPT // ZROWS)
    def _(j):
        pltpu.sync_copy(zbuf, acc_sh.at[pl.ds(sid * RPT + j * ZROWS, ZROWS)])

    plsc.subcore_barrier()

    def issue(c):
        b = lax.rem(c, NBUF)
        base = w * EPW + c * G
        pltpu.sync_copy(src_hbm.at[pl.ds(base, G)], srcv.at[b])
        pltpu.sync_copy(dst_hbm.at[pl.ds(base, G)], dstv.at[b])
        pltpu.async_copy(x_hbm.at[srcv.at[b]], rowsv.at[pl.ds(b * G, G)],
                         sems.at[b])

    def drain(c):
        b = lax.rem(c, NBUF)
        pltpu.make_async_copy(x_hbm.at[srcv.at[b]], rowsv.at[pl.ds(b * G, G)],
                              sems.at[b]).wait()
        pltpu.sync_copy(rowsv.at[pl.ds(b * G, G)], acc_sh.at[dstv.at[b]],
                        add=True)

    @pl.loop(0, NBUF - 1)
    def _(c):
        issue(c)

    @pl.loop(0, CHUNKS)
    def _(c):
        drain(c)

        @pl.when(c + NBUF - 1 < CHUNKS)
        def _():
            issue(c + NBUF - 1)

    plsc.subcore_barrier()

    r0 = sid * RPT
    pltpu.sync_copy(acc_sh.at[pl.ds(r0, RPT)], out_hbm.at[cid].at[pl.ds(r0, RPT)])


def _sc_aggregate(x, src, dst):
    k = pl.kernel(
        _agg_body,
        out_type=jax.ShapeDtypeStruct((NC, NPAD, D), jnp.float32),
        mesh=_mesh(),
        scratch_types=[
            pltpu.VMEM((NBUF, G), jnp.int32),
            pltpu.VMEM((NBUF, G), jnp.int32),
            pltpu.VMEM((NBUF * G, D), jnp.float32),
            pltpu.VMEM((ZROWS, D), jnp.float32),
            pltpu.VMEM_SHARED((NPAD, D), jnp.float32),
            pltpu.SemaphoreType.DMA((NBUF,)),
        ],
    )
    return k(x, src, dst)


def _deg_body(dst_hbm, deg_hbm, dstv0, dstv1, onesv, zdeg, deg_sh, sem0, sem1):
    cid = lax.axis_index("c")
    sid = lax.axis_index("s")
    w = cid * NS + sid

    _fill_rows(zdeg, ZROWS, D, 0.0)
    _fill_rows(onesv, G, D, 1.0)

    @pl.loop(0, RPT // ZROWS)
    def _(j):
        pltpu.sync_copy(zdeg, deg_sh.at[pl.ds(sid * RPT + j * ZROWS, ZROWS)])

    plsc.subcore_barrier()

    def load_idx(c, dstv, sem):
        pltpu.async_copy(dst_hbm.at[pl.ds(w * EPW + c * G, G)], dstv, sem)

    def wait_and_scatter(c, dstv, sem):
        pltpu.make_async_copy(dst_hbm.at[pl.ds(w * EPW + c * G, G)], dstv,
                              sem).wait()
        pltpu.sync_copy(onesv, deg_sh.at[dstv], add=True)

    load_idx(0, dstv0, sem0)
    load_idx(1, dstv1, sem1)

    @pl.loop(0, (CHUNKS - 1) // 2)
    def _(j):
        wait_and_scatter(2 * j, dstv0, sem0)
        load_idx(2 * j + 2, dstv0, sem0)
        wait_and_scatter(2 * j + 1, dstv1, sem1)

        @pl.when(j < (CHUNKS - 1) // 2 - 1)
        def _():
            load_idx(2 * j + 3, dstv1, sem1)

    wait_and_scatter(CHUNKS - 1, dstv0, sem0)

    plsc.subcore_barrier()

    r0 = sid * RPT
    pltpu.sync_copy(deg_sh.at[pl.ds(r0, RPT)], deg_hbm.at[cid].at[pl.ds(r0, RPT)])


def _sc_degrees(dst):
    k = pl.kernel(
        _deg_body,
        out_type=jax.ShapeDtypeStruct((NC, NPAD, D), jnp.float32),
        mesh=_mesh(),
        scratch_types=[
            pltpu.VMEM((G,), jnp.int32),
            pltpu.VMEM((G,), jnp.int32),
            pltpu.VMEM((G, D), jnp.float32),
            pltpu.VMEM((ZROWS, D), jnp.float32),
            pltpu.VMEM_SHARED((NPAD, D), jnp.float32),
            pltpu.SemaphoreType.DMA,
            pltpu.SemaphoreType.DMA,
        ],
    )
    return k(dst)


def _mask_features(mask_f, feat):
    def body(m_ref, x_ref, o_ref):
        o_ref[...] = jnp.where(m_ref[...] > 0.0, 0.0, x_ref[...])

    return pl.pallas_call(
        body, out_shape=jax.ShapeDtypeStruct((N, D), jnp.float32)
    )(mask_f, feat)


_BR = 1000


def _neigh_mean(agg_ref, deg_ref):
    agg = agg_ref[0] + agg_ref[1]
    deg = deg_ref[0, :, 0:1] + deg_ref[1, :, 0:1]
    return agg / jnp.maximum(deg, 1.0)


def _sage_layer(x_self, agg_p, deg_p, w_self, w_neigh, b, g_w, g_b, relu):
    def body(xs_ref, agg_ref, deg_ref, ws_ref, wn_ref, b_ref, gw_ref, gb_ref, o_ref):
        h = _neigh_mean(agg_ref, deg_ref)
        z = (
            jnp.dot(xs_ref[...], ws_ref[...], preferred_element_type=jnp.float32)
            + jnp.dot(h, wn_ref[...], preferred_element_type=jnp.float32)
            + b_ref[...]
        )
        gate = jax.nn.sigmoid(
            jnp.dot(z, gw_ref[...], preferred_element_type=jnp.float32) + gb_ref[...]
        )
        z = z * gate
        if relu:
            z = jnp.maximum(z, 0.0)
        o_ref[...] = z

    wspec = pl.BlockSpec((D, D), lambda i: (0, 0))
    bspec = pl.BlockSpec((1, D), lambda i: (0, 0))
    return pl.pallas_call(
        body,
        grid=(N // _BR,),
        in_specs=[
            pl.BlockSpec((_BR, D), lambda i: (i, 0)),
            pl.BlockSpec((NC, _BR, D), lambda i: (0, i, 0)),
            pl.BlockSpec((NC, _BR, D), lambda i: (0, i, 0)),
            wspec, wspec, bspec, wspec, bspec,
        ],
        out_specs=pl.BlockSpec((_BR, D), lambda i: (i, 0)),
        out_shape=jax.ShapeDtypeStruct((N, D), jnp.float32),
    )(x_self, agg_p, deg_p, w_self, w_neigh, b, g_w, g_b)


def _final_layer(z1, agg_p, deg_p, w_self, w_neigh, b, g_w, g_b, dec_w, dec_b,
                 feat, mask_f):
    nblk = N // _BR

    def body(z1_ref, agg_ref, deg_ref, ws_ref, wn_ref, b_ref, gw_ref, gb_ref,
             dw_ref, db_ref, feat_ref, m_ref, o_ref, loss_ref, acc):
        i = pl.program_id(0)

        @pl.when(i == 0)
        def _():
            acc[0] = 0.0
            acc[1] = 0.0

        h = _neigh_mean(agg_ref, deg_ref)
        z = (
            jnp.dot(z1_ref[...], ws_ref[...], preferred_element_type=jnp.float32)
            + jnp.dot(h, wn_ref[...], preferred_element_type=jnp.float32)
            + b_ref[...]
        )
        gate = jax.nn.sigmoid(
            jnp.dot(z, gw_ref[...], preferred_element_type=jnp.float32) + gb_ref[...]
        )
        z = z * gate
        o_ref[...] = z

        rec = jnp.dot(z, dw_ref[...], preferred_element_type=jnp.float32) + db_ref[...]
        rec_n = rec / jnp.maximum(
            jnp.sqrt(jnp.sum(rec * rec, axis=1, keepdims=True)), 1e-12
        )
        feat_b = feat_ref[...]
        orig_n = feat_b / jnp.maximum(
            jnp.sqrt(jnp.sum(feat_b * feat_b, axis=1, keepdims=True)), 1e-12
        )
        cos = jnp.sum(rec_n * orig_n, axis=1, keepdims=True)
        scaling = jnp.maximum(
            jnp.sqrt(jnp.sum(rec_n * rec_n, axis=1, keepdims=True)), 1e-6
        )
        err = 1.0 - cos / scaling
        m = jnp.max(m_ref[...], axis=1, keepdims=True)
        acc[0] += jnp.sum(err * m)
        acc[1] += jnp.sum(m)

        @pl.when(i == nblk - 1)
        def _():
            loss_ref[0, 0] = acc[0] / jnp.maximum(acc[1], 1.0)

    wspec = pl.BlockSpec((D, D), lambda i: (0, 0))
    bspec = pl.BlockSpec((1, D), lambda i: (0, 0))
    return pl.pallas_call(
        body,
        grid=(nblk,),
        in_specs=[
            pl.BlockSpec((_BR, D), lambda i: (i, 0)),
            pl.BlockSpec((NC, _BR, D), lambda i: (0, i, 0)),
            pl.BlockSpec((NC, _BR, D), lambda i: (0, i, 0)),
            wspec, wspec, bspec, wspec, bspec, wspec, bspec,
            pl.BlockSpec((_BR, D), lambda i: (i, 0)),
            pl.BlockSpec((_BR, D), lambda i: (i, 0)),
        ],
        out_specs=[
            pl.BlockSpec((_BR, D), lambda i: (i, 0)),
            pl.BlockSpec(memory_space=pltpu.SMEM),
        ],
        out_shape=[
            jax.ShapeDtypeStruct((N, D), jnp.float32),
            jax.ShapeDtypeStruct((1, 1), jnp.float32),
        ],
        scratch_shapes=[pltpu.SMEM((2,), jnp.float32)],
    )(z1, agg_p, deg_p, w_self, w_neigh, b, g_w, g_b, dec_w, dec_b, feat, mask_f)


def kernel(in_feat, edge_index, W1_self, W1_neigh, b1, W2_self, W2_neigh, b2,
           G1_W, G1_b, G2_W, G2_b, Dec_W, Dec_b):
    mask_f = jnp.asarray(_MASK_F32)
    src = edge_index[0].astype(jnp.int32)
    dst = edge_index[1].astype(jnp.int32)

    masked = _mask_features(mask_f, in_feat)
    deg_p = _sc_degrees(dst)
    agg1_p = _sc_aggregate(masked, src, dst)
    z1 = _sage_layer(
        masked, agg1_p, deg_p,
        W1_self, W1_neigh, b1.reshape(1, D), G1_W, G1_b.reshape(1, D), relu=True,
    )
    agg2_p = _sc_aggregate(z1, src, dst)
    z, loss = _final_layer(
        z1, agg2_p, deg_p,
        W2_self, W2_neigh, b2.reshape(1, D), G2_W, G2_b.reshape(1, D),
        Dec_W, Dec_b.reshape(1, D), in_feat, mask_f,
    )
    return z, loss[0, 0]


# speedup vs baseline: 1.4897x; 1.4897x over previous
import jax
import jax.numpy as jnp
import numpy as np
from jax import lax
from jax.experimental import pallas as pl
from jax.experimental.pallas import tpu as pltpu
from jax.experimental.pallas import tpu_sc as plsc

N = 10000
E = 320000
D = 128
MASK_RATIO = 0.5

NC = 2
NS = 16
NW = NC * NS
EPW = E // NW
G = 80
CHUNKS = EPW // G
NPAD = 10240
RPT = NPAD // NS
NBUF = 3
ZROWS = 128

with jax.default_device(jax.devices("cpu")[0]):
    _MASK_F32 = np.asarray(
        jax.random.uniform(jax.random.key(42), (N, D)) < MASK_RATIO,
        dtype=np.float32,
    )


def _mesh():
    return plsc.VectorSubcoreMesh(
        core_axis_name="c", subcore_axis_name="s", num_cores=NC, num_subcores=NS
    )


def _fill_rows(ref, nrows, ncols, value):
    vec = jnp.full((16,), value, jnp.float32)

    @pl.loop(0, nrows)
    def _(r):
        @pl.loop(0, ncols // 16)
        def _(q):
            ref[r, pl.ds(q * 16, 16)] = vec


def _agg_body(x_hbm, src_hbm, dst_hbm, out_hbm, srcv, dstv, rowsv, zbuf,
              acc_sh, sem_g, sem_i, sem_s):
    cid = lax.axis_index("c")
    sid = lax.axis_index("s")
    w = cid * NS + sid

    _fill_rows(zbuf, ZROWS, D, 0.0)

    @pl.loop(0, RPT // ZROWS)
    def _(j):
        pltpu.sync_copy(zbuf, acc_sh.at[pl.ds(sid * RPT + j * ZROWS, ZROWS)])

    plsc.subcore_barrier()

    def idx_descs(c, b):
        base = w * EPW + c * G
        return (
            pltpu.make_async_copy(src_hbm.at[pl.ds(base, G)], srcv.at[b],
                                  sem_i.at[b]),
            pltpu.make_async_copy(dst_hbm.at[pl.ds(base, G)], dstv.at[b],
                                  sem_i.at[b]),
        )

    def issue_idx(c):
        b = lax.rem(c, NBUF)
        d1, d2 = idx_descs(c, b)
        d1.start()
        d2.start()

    def gather_desc(b):
        return pltpu.make_async_copy(x_hbm.at[srcv.at[b]],
                                     rowsv.at[pl.ds(b * G, G)], sem_g.at[b])

    def scatter_desc(b):
        return pltpu.make_async_copy(rowsv.at[pl.ds(b * G, G)],
                                     acc_sh.at[dstv.at[b]], sem_s.at[b])

    @pl.loop(0, NBUF)
    def _(c):
        issue_idx(c)

    @pl.loop(0, NBUF - 1)
    def _(c):
        d1, d2 = idx_descs(c, c)
        d1.wait()
        d2.wait()
        gather_desc(c).start()

    @pl.loop(0, CHUNKS)
    def _(c):
        b = lax.rem(c, NBUF)
        gather_desc(b).wait()
        scatter_desc(b).start(add=True)

        @pl.when(c + NBUF < CHUNKS)
        def _():
            issue_idx(c + NBUF)

        @pl.when(c + NBUF - 1 < CHUNKS)
        def _():
            b3 = lax.rem(c + NBUF - 1, NBUF)
            d1, d2 = idx_descs(c + NBUF - 1, b3)
            d1.wait()
            d2.wait()

            @pl.when(c >= 1)
            def _():
                scatter_desc(b3).wait()

            gather_desc(b3).start()

    @pl.loop(CHUNKS - NBUF, CHUNKS)
    def _(c):
        scatter_desc(lax.rem(c, NBUF)).wait()

    plsc.subcore_barrier()

    r0 = sid * RPT
    pltpu.sync_copy(acc_sh.at[pl.ds(r0, RPT)], out_hbm.at[cid].at[pl.ds(r0, RPT)])


def _sc_aggregate(x, src, dst):
    k = pl.kernel(
        _agg_body,
        out_type=jax.ShapeDtypeStruct((NC, NPAD, D), jnp.float32),
        mesh=_mesh(),
        scratch_types=[
            pltpu.VMEM((NBUF, G), jnp.int32),
            pltpu.VMEM((NBUF, G), jnp.int32),
            pltpu.VMEM((NBUF * G, D), jnp.float32),
            pltpu.VMEM((ZROWS, D), jnp.float32),
            pltpu.VMEM_SHARED((NPAD, D), jnp.float32),
            pltpu.SemaphoreType.DMA((NBUF,)),
            pltpu.SemaphoreType.DMA((NBUF,)),
            pltpu.SemaphoreType.DMA((NBUF,)),
        ],
    )
    return k(x, src, dst)


def _deg_body(dst_hbm, deg_hbm, dstv0, dstv1, onesv, zdeg, deg_sh, sem0, sem1):
    cid = lax.axis_index("c")
    sid = lax.axis_index("s")
    w = cid * NS + sid

    _fill_rows(zdeg, ZROWS, D, 0.0)
    _fill_rows(onesv, G, D, 1.0)

    @pl.loop(0, RPT // ZROWS)
    def _(j):
        pltpu.sync_copy(zdeg, deg_sh.at[pl.ds(sid * RPT + j * ZROWS, ZROWS)])

    plsc.subcore_barrier()

    def load_idx(c, dstv, sem):
        pltpu.async_copy(dst_hbm.at[pl.ds(w * EPW + c * G, G)], dstv, sem)

    def wait_and_scatter(c, dstv, sem):
        pltpu.make_async_copy(dst_hbm.at[pl.ds(w * EPW + c * G, G)], dstv,
                              sem).wait()
        pltpu.sync_copy(onesv, deg_sh.at[dstv], add=True)

    load_idx(0, dstv0, sem0)
    load_idx(1, dstv1, sem1)

    @pl.loop(0, (CHUNKS - 1) // 2)
    def _(j):
        wait_and_scatter(2 * j, dstv0, sem0)
        load_idx(2 * j + 2, dstv0, sem0)
        wait_and_scatter(2 * j + 1, dstv1, sem1)

        @pl.when(j < (CHUNKS - 1) // 2 - 1)
        def _():
            load_idx(2 * j + 3, dstv1, sem1)

    wait_and_scatter(CHUNKS - 1, dstv0, sem0)

    plsc.subcore_barrier()

    r0 = sid * RPT
    pltpu.sync_copy(deg_sh.at[pl.ds(r0, RPT)], deg_hbm.at[cid].at[pl.ds(r0, RPT)])


def _sc_degrees(dst):
    k = pl.kernel(
        _deg_body,
        out_type=jax.ShapeDtypeStruct((NC, NPAD, D), jnp.float32),
        mesh=_mesh(),
        scratch_types=[
            pltpu.VMEM((G,), jnp.int32),
            pltpu.VMEM((G,), jnp.int32),
            pltpu.VMEM((G, D), jnp.float32),
            pltpu.VMEM((ZROWS, D), jnp.float32),
            pltpu.VMEM_SHARED((NPAD, D), jnp.float32),
            pltpu.SemaphoreType.DMA,
            pltpu.SemaphoreType.DMA,
        ],
    )
    return k(dst)


def _mask_features(mask_f, feat):
    def body(m_ref, x_ref, o_ref):
        o_ref[...] = jnp.where(m_ref[...] > 0.0, 0.0, x_ref[...])

    return pl.pallas_call(
        body, out_shape=jax.ShapeDtypeStruct((N, D), jnp.float32)
    )(mask_f, feat)


_BR = 1000


def _neigh_mean(agg_ref, deg_ref):
    agg = agg_ref[0] + agg_ref[1]
    deg = deg_ref[0, :, 0:1] + deg_ref[1, :, 0:1]
    return agg / jnp.maximum(deg, 1.0)


def _sage_layer(x_self, agg_p, deg_p, w_self, w_neigh, b, g_w, g_b, relu):
    def body(xs_ref, agg_ref, deg_ref, ws_ref, wn_ref, b_ref, gw_ref, gb_ref, o_ref):
        h = _neigh_mean(agg_ref, deg_ref)
        z = (
            jnp.dot(xs_ref[...], ws_ref[...], preferred_element_type=jnp.float32)
            + jnp.dot(h, wn_ref[...], preferred_element_type=jnp.float32)
            + b_ref[...]
        )
        gate = jax.nn.sigmoid(
            jnp.dot(z, gw_ref[...], preferred_element_type=jnp.float32) + gb_ref[...]
        )
        z = z * gate
        if relu:
            z = jnp.maximum(z, 0.0)
        o_ref[...] = z

    wspec = pl.BlockSpec((D, D), lambda i: (0, 0))
    bspec = pl.BlockSpec((1, D), lambda i: (0, 0))
    return pl.pallas_call(
        body,
        grid=(N // _BR,),
        in_specs=[
            pl.BlockSpec((_BR, D), lambda i: (i, 0)),
            pl.BlockSpec((NC, _BR, D), lambda i: (0, i, 0)),
            pl.BlockSpec((NC, _BR, D), lambda i: (0, i, 0)),
            wspec, wspec, bspec, wspec, bspec,
        ],
        out_specs=pl.BlockSpec((_BR, D), lambda i: (i, 0)),
        out_shape=jax.ShapeDtypeStruct((N, D), jnp.float32),
    )(x_self, agg_p, deg_p, w_self, w_neigh, b, g_w, g_b)


def _final_layer(z1, agg_p, deg_p, w_self, w_neigh, b, g_w, g_b, dec_w, dec_b,
                 feat, mask_f):
    nblk = N // _BR

    def body(z1_ref, agg_ref, deg_ref, ws_ref, wn_ref, b_ref, gw_ref, gb_ref,
             dw_ref, db_ref, feat_ref, m_ref, o_ref, loss_ref, acc):
        i = pl.program_id(0)

        @pl.when(i == 0)
        def _():
            acc[0] = 0.0
            acc[1] = 0.0

        h = _neigh_mean(agg_ref, deg_ref)
        z = (
            jnp.dot(z1_ref[...], ws_ref[...], preferred_element_type=jnp.float32)
            + jnp.dot(h, wn_ref[...], preferred_element_type=jnp.float32)
            + b_ref[...]
        )
        gate = jax.nn.sigmoid(
            jnp.dot(z, gw_ref[...], preferred_element_type=jnp.float32) + gb_ref[...]
        )
        z = z * gate
        o_ref[...] = z

        rec = jnp.dot(z, dw_ref[...], preferred_element_type=jnp.float32) + db_ref[...]
        rec_n = rec / jnp.maximum(
            jnp.sqrt(jnp.sum(rec * rec, axis=1, keepdims=True)), 1e-12
        )
        feat_b = feat_ref[...]
        orig_n = feat_b / jnp.maximum(
            jnp.sqrt(jnp.sum(feat_b * feat_b, axis=1, keepdims=True)), 1e-12
        )
        cos = jnp.sum(rec_n * orig_n, axis=1, keepdims=True)
        scaling = jnp.maximum(
            jnp.sqrt(jnp.sum(rec_n * rec_n, axis=1, keepdims=True)), 1e-6
        )
        err = 1.0 - cos / scaling
        m = jnp.max(m_ref[...], axis=1, keepdims=True)
        acc[0] += jnp.sum(err * m)
        acc[1] += jnp.sum(m)

        @pl.when(i == nblk - 1)
        def _():
            loss_ref[0, 0] = acc[0] / jnp.maximum(acc[1], 1.0)

    wspec = pl.BlockSpec((D, D), lambda i: (0, 0))
    bspec = pl.BlockSpec((1, D), lambda i: (0, 0))
    return pl.pallas_call(
        body,
        grid=(nblk,),
        in_specs=[
            pl.BlockSpec((_BR, D), lambda i: (i, 0)),
            pl.BlockSpec((NC, _BR, D), lambda i: (0, i, 0)),
            pl.BlockSpec((NC, _BR, D), lambda i: (0, i, 0)),
            wspec, wspec, bspec, wspec, bspec, wspec, bspec,
            pl.BlockSpec((_BR, D), lambda i: (i, 0)),
            pl.BlockSpec((_BR, D), lambda i: (i, 0)),
        ],
        out_specs=[
            pl.BlockSpec((_BR, D), lambda i: (i, 0)),
            pl.BlockSpec(memory_space=pltpu.SMEM),
        ],
        out_shape=[
            jax.ShapeDtypeStruct((N, D), jnp.float32),
            jax.ShapeDtypeStruct((1, 1), jnp.float32),
        ],
        scratch_shapes=[pltpu.SMEM((2,), jnp.float32)],
    )(z1, agg_p, deg_p, w_self, w_neigh, b, g_w, g_b, dec_w, dec_b, feat, mask_f)


def kernel(in_feat, edge_index, W1_self, W1_neigh, b1, W2_self, W2_neigh, b2,
           G1_W, G1_b, G2_W, G2_b, Dec_W, Dec_b):
    mask_f = jnp.asarray(_MASK_F32)
    src = edge_index[0].astype(jnp.int32)
    dst = edge_index[1].astype(jnp.int32)

    masked = _mask_features(mask_f, in_feat)
    deg_p = _sc_degrees(dst)
    agg1_p = _sc_aggregate(masked, src, dst)
    z1 = _sage_layer(
        masked, agg1_p, deg_p,
        W1_self, W1_neigh, b1.reshape(1, D), G1_W, G1_b.reshape(1, D), relu=True,
    )
    agg2_p = _sc_aggregate(z1, src, dst)
    z, loss = _final_layer(
        z1, agg2_p, deg_p,
        W2_self, W2_neigh, b2.reshape(1, D), G2_W, G2_b.reshape(1, D),
        Dec_W, Dec_b.reshape(1, D), in_feat, mask_f,
    )
    return z, loss[0, 0]


# speedup vs baseline: 1.5047x; 1.0100x over previous
import jax
import jax.numpy as jnp
import numpy as np
from jax import lax
from jax.experimental import pallas as pl
from jax.experimental.pallas import tpu as pltpu
from jax.experimental.pallas import tpu_sc as plsc

N = 10000
E = 320000
D = 128
MASK_RATIO = 0.5

NC = 2
NS = 16
NW = NC * NS
EPW = E // NW
G = 80
CHUNKS = EPW // G
NPAD = 10240
RPT = NPAD // NS
NBUF = 3
ZROWS = 128

with jax.default_device(jax.devices("cpu")[0]):
    _MASK_F32 = np.asarray(
        jax.random.uniform(jax.random.key(42), (N, D)) < MASK_RATIO,
        dtype=np.float32,
    )


def _mesh():
    return plsc.VectorSubcoreMesh(
        core_axis_name="c", subcore_axis_name="s", num_cores=NC, num_subcores=NS
    )


def _fill_rows(ref, nrows, ncols, value):
    vec = jnp.full((16,), value, jnp.float32)

    @pl.loop(0, nrows)
    def _(r):
        @pl.loop(0, ncols // 16)
        def _(q):
            ref[r, pl.ds(q * 16, 16)] = vec


def _agg_body(x_hbm, src_hbm, dst_hbm, out_hbm, srcv, dstv, rowsv, zbuf,
              acc_sh, sem_g, sem_i, sem_s):
    cid = lax.axis_index("c")
    sid = lax.axis_index("s")
    w = cid * NS + sid

    _fill_rows(zbuf, ZROWS, D, 0.0)

    @pl.loop(0, RPT // ZROWS)
    def _(j):
        pltpu.sync_copy(zbuf, acc_sh.at[pl.ds(sid * RPT + j * ZROWS, ZROWS)])

    plsc.subcore_barrier()

    def idx_descs(c, b):
        base = w * EPW + c * G
        return (
            pltpu.make_async_copy(src_hbm.at[pl.ds(base, G)], srcv.at[b],
                                  sem_i.at[b]),
            pltpu.make_async_copy(dst_hbm.at[pl.ds(base, G)], dstv.at[b],
                                  sem_i.at[b]),
        )

    def issue_idx(c):
        b = lax.rem(c, NBUF)
        d1, d2 = idx_descs(c, b)
        d1.start()
        d2.start()

    def gather_desc(b):
        return pltpu.make_async_copy(x_hbm.at[srcv.at[b]],
                                     rowsv.at[pl.ds(b * G, G)], sem_g.at[b])

    def scatter_desc(b):
        return pltpu.make_async_copy(rowsv.at[pl.ds(b * G, G)],
                                     acc_sh.at[dstv.at[b]], sem_s.at[b])

    @pl.loop(0, NBUF)
    def _(c):
        issue_idx(c)

    @pl.loop(0, NBUF - 1)
    def _(c):
        d1, d2 = idx_descs(c, c)
        d1.wait()
        d2.wait()
        gather_desc(c).start()

    @pl.loop(0, CHUNKS)
    def _(c):
        b = lax.rem(c, NBUF)
        gather_desc(b).wait()
        scatter_desc(b).start(add=True)

        @pl.when(c + NBUF < CHUNKS)
        def _():
            issue_idx(c + NBUF)

        @pl.when(c + NBUF - 1 < CHUNKS)
        def _():
            b3 = lax.rem(c + NBUF - 1, NBUF)
            d1, d2 = idx_descs(c + NBUF - 1, b3)
            d1.wait()
            d2.wait()

            @pl.when(c >= 1)
            def _():
                scatter_desc(b3).wait()

            gather_desc(b3).start()

    @pl.loop(CHUNKS - NBUF, CHUNKS)
    def _(c):
        scatter_desc(lax.rem(c, NBUF)).wait()

    plsc.subcore_barrier()

    r0 = sid * RPT
    pltpu.sync_copy(acc_sh.at[pl.ds(r0, RPT)], out_hbm.at[cid].at[pl.ds(r0, RPT)])


def _sc_aggregate(x, src, dst):
    k = pl.kernel(
        _agg_body,
        out_type=jax.ShapeDtypeStruct((NC, NPAD, D), jnp.float32),
        mesh=_mesh(),
        scratch_types=[
            pltpu.VMEM((NBUF, G), jnp.int32),
            pltpu.VMEM((NBUF, G), jnp.int32),
            pltpu.VMEM((NBUF * G, D), jnp.float32),
            pltpu.VMEM((ZROWS, D), jnp.float32),
            pltpu.VMEM_SHARED((NPAD, D), jnp.float32),
            pltpu.SemaphoreType.DMA((NBUF,)),
            pltpu.SemaphoreType.DMA((NBUF,)),
            pltpu.SemaphoreType.DMA((NBUF,)),
        ],
    )
    return k(x, src, dst)


def _deg_body(dst_hbm, deg_hbm, dstv, onesv, zdeg, deg_sh, sem_i, sem_s):
    cid = lax.axis_index("c")
    sid = lax.axis_index("s")
    w = cid * NS + sid

    _fill_rows(zdeg, ZROWS, D, 0.0)
    _fill_rows(onesv, G, D, 1.0)

    @pl.loop(0, RPT // ZROWS)
    def _(j):
        pltpu.sync_copy(zdeg, deg_sh.at[pl.ds(sid * RPT + j * ZROWS, ZROWS)])

    plsc.subcore_barrier()

    def idx_desc(c, b):
        return pltpu.make_async_copy(dst_hbm.at[pl.ds(w * EPW + c * G, G)],
                                     dstv.at[b], sem_i.at[b])

    def scatter_desc(b):
        return pltpu.make_async_copy(onesv, deg_sh.at[dstv.at[b]], sem_s.at[b])

    @pl.loop(0, NBUF)
    def _(c):
        idx_desc(c, c).start()

    @pl.loop(0, CHUNKS)
    def _(c):
        b = lax.rem(c, NBUF)
        idx_desc(c, b).wait()
        scatter_desc(b).start(add=True)

        @pl.when((c >= 1) & (c + NBUF - 1 < CHUNKS))
        def _():
            b3 = lax.rem(c - 1, NBUF)
            scatter_desc(b3).wait()
            idx_desc(c + NBUF - 1, b3).start()

    @pl.loop(CHUNKS - NBUF, CHUNKS)
    def _(c):
        scatter_desc(lax.rem(c, NBUF)).wait()

    plsc.subcore_barrier()

    r0 = sid * RPT
    pltpu.sync_copy(deg_sh.at[pl.ds(r0, RPT)], deg_hbm.at[cid].at[pl.ds(r0, RPT)])


def _sc_degrees(dst):
    k = pl.kernel(
        _deg_body,
        out_type=jax.ShapeDtypeStruct((NC, NPAD, D), jnp.float32),
        mesh=_mesh(),
        scratch_types=[
            pltpu.VMEM((NBUF, G), jnp.int32),
            pltpu.VMEM((G, D), jnp.float32),
            pltpu.VMEM((ZROWS, D), jnp.float32),
            pltpu.VMEM_SHARED((NPAD, D), jnp.float32),
            pltpu.SemaphoreType.DMA((NBUF,)),
            pltpu.SemaphoreType.DMA((NBUF,)),
        ],
    )
    return k(dst)


def _mask_features(mask_f, feat):
    def body(m_ref, x_ref, o_ref):
        o_ref[...] = jnp.where(m_ref[...] > 0.0, 0.0, x_ref[...])

    return pl.pallas_call(
        body, out_shape=jax.ShapeDtypeStruct((N, D), jnp.float32)
    )(mask_f, feat)


_BR = 1000


def _neigh_mean(agg_ref, deg_ref):
    agg = agg_ref[0] + agg_ref[1]
    deg = deg_ref[0, :, 0:1] + deg_ref[1, :, 0:1]
    return agg / jnp.maximum(deg, 1.0)


def _sage_layer(x_self, agg_p, deg_p, w_self, w_neigh, b, g_w, g_b, relu):
    def body(xs_ref, agg_ref, deg_ref, ws_ref, wn_ref, b_ref, gw_ref, gb_ref, o_ref):
        h = _neigh_mean(agg_ref, deg_ref)
        z = (
            jnp.dot(xs_ref[...], ws_ref[...], preferred_element_type=jnp.float32)
            + jnp.dot(h, wn_ref[...], preferred_element_type=jnp.float32)
            + b_ref[...]
        )
        gate = jax.nn.sigmoid(
            jnp.dot(z, gw_ref[...], preferred_element_type=jnp.float32) + gb_ref[...]
        )
        z = z * gate
        if relu:
            z = jnp.maximum(z, 0.0)
        o_ref[...] = z

    wspec = pl.BlockSpec((D, D), lambda i: (0, 0))
    bspec = pl.BlockSpec((1, D), lambda i: (0, 0))
    return pl.pallas_call(
        body,
        grid=(N // _BR,),
        in_specs=[
            pl.BlockSpec((_BR, D), lambda i: (i, 0)),
            pl.BlockSpec((NC, _BR, D), lambda i: (0, i, 0)),
            pl.BlockSpec((NC, _BR, D), lambda i: (0, i, 0)),
            wspec, wspec, bspec, wspec, bspec,
        ],
        out_specs=pl.BlockSpec((_BR, D), lambda i: (i, 0)),
        out_shape=jax.ShapeDtypeStruct((N, D), jnp.float32),
    )(x_self, agg_p, deg_p, w_self, w_neigh, b, g_w, g_b)


def _final_layer(z1, agg_p, deg_p, w_self, w_neigh, b, g_w, g_b, dec_w, dec_b,
                 feat, mask_f):
    nblk = N // _BR

    def body(z1_ref, agg_ref, deg_ref, ws_ref, wn_ref, b_ref, gw_ref, gb_ref,
             dw_ref, db_ref, feat_ref, m_ref, o_ref, loss_ref, acc):
        i = pl.program_id(0)

        @pl.when(i == 0)
        def _():
            acc[0] = 0.0
            acc[1] = 0.0

        h = _neigh_mean(agg_ref, deg_ref)
        z = (
            jnp.dot(z1_ref[...], ws_ref[...], preferred_element_type=jnp.float32)
            + jnp.dot(h, wn_ref[...], preferred_element_type=jnp.float32)
            + b_ref[...]
        )
        gate = jax.nn.sigmoid(
            jnp.dot(z, gw_ref[...], preferred_element_type=jnp.float32) + gb_ref[...]
        )
        z = z * gate
        o_ref[...] = z

        rec = jnp.dot(z, dw_ref[...], preferred_element_type=jnp.float32) + db_ref[...]
        rec_n = rec / jnp.maximum(
            jnp.sqrt(jnp.sum(rec * rec, axis=1, keepdims=True)), 1e-12
        )
        feat_b = feat_ref[...]
        orig_n = feat_b / jnp.maximum(
            jnp.sqrt(jnp.sum(feat_b * feat_b, axis=1, keepdims=True)), 1e-12
        )
        cos = jnp.sum(rec_n * orig_n, axis=1, keepdims=True)
        scaling = jnp.maximum(
            jnp.sqrt(jnp.sum(rec_n * rec_n, axis=1, keepdims=True)), 1e-6
        )
        err = 1.0 - cos / scaling
        m = jnp.max(m_ref[...], axis=1, keepdims=True)
        acc[0] += jnp.sum(err * m)
        acc[1] += jnp.sum(m)

        @pl.when(i == nblk - 1)
        def _():
            loss_ref[0, 0] = acc[0] / jnp.maximum(acc[1], 1.0)

    wspec = pl.BlockSpec((D, D), lambda i: (0, 0))
    bspec = pl.BlockSpec((1, D), lambda i: (0, 0))
    return pl.pallas_call(
        body,
        grid=(nblk,),
        in_specs=[
            pl.BlockSpec((_BR, D), lambda i: (i, 0)),
            pl.BlockSpec((NC, _BR, D), lambda i: (0, i, 0)),
            pl.BlockSpec((NC, _BR, D), lambda i: (0, i, 0)),
            wspec, wspec, bspec, wspec, bspec, wspec, bspec,
            pl.BlockSpec((_BR, D), lambda i: (i, 0)),
            pl.BlockSpec((_BR, D), lambda i: (i, 0)),
        ],
        out_specs=[
            pl.BlockSpec((_BR, D), lambda i: (i, 0)),
            pl.BlockSpec(memory_space=pltpu.SMEM),
        ],
        out_shape=[
            jax.ShapeDtypeStruct((N, D), jnp.float32),
            jax.ShapeDtypeStruct((1, 1), jnp.float32),
        ],
        scratch_shapes=[pltpu.SMEM((2,), jnp.float32)],
    )(z1, agg_p, deg_p, w_self, w_neigh, b, g_w, g_b, dec_w, dec_b, feat, mask_f)


def kernel(in_feat, edge_index, W1_self, W1_neigh, b1, W2_self, W2_neigh, b2,
           G1_W, G1_b, G2_W, G2_b, Dec_W, Dec_b):
    mask_f = jnp.asarray(_MASK_F32)
    src = edge_index[0].astype(jnp.int32)
    dst = edge_index[1].astype(jnp.int32)

    masked = _mask_features(mask_f, in_feat)
    deg_p = _sc_degrees(dst)
    agg1_p = _sc_aggregate(masked, src, dst)
    z1 = _sage_layer(
        masked, agg1_p, deg_p,
        W1_self, W1_neigh, b1.reshape(1, D), G1_W, G1_b.reshape(1, D), relu=True,
    )
    agg2_p = _sc_aggregate(z1, src, dst)
    z, loss = _final_layer(
        z1, agg2_p, deg_p,
        W2_self, W2_neigh, b2.reshape(1, D), G2_W, G2_b.reshape(1, D),
        Dec_W, Dec_b.reshape(1, D), in_feat, mask_f,
    )
    return z, loss[0, 0]
